# Initial kernel scaffold; baseline (speedup 1.0000x reference)
import jax
import jax.numpy as jnp
from jax.experimental import pallas as pl


def kernel(rand_attn, noise_attn, rand_kqv, noise_k, noise_q, noise_v, perturb_attn, perturb_k, perturb_q, perturb_v):
    raise NotImplementedError("write your pallas kernel here")


# speedup vs baseline: 34.6577x; 34.6577x over previous
import functools

import jax
import jax.numpy as jnp
from jax.experimental import pallas as pl

N_LAYERS = 32
N_HEADS = 32
BSZ = 4096
K = 256
TOTAL = N_LAYERS * N_HEADS

_SCALE = 16777216.0
_BITS = 24


def _row_threshold(m, k):
    rows = m.shape[0]
    lo = jnp.zeros((rows, 1), dtype=jnp.int32)
    hi = jnp.full((rows, 1), 1 << _BITS, dtype=jnp.int32)

    def body(_, carry):
        lo, hi = carry
        mid = (lo + hi) >> 1
        cnt = jnp.sum((m >= mid).astype(jnp.float32), axis=1, keepdims=True)
        pick = cnt >= float(k)
        lo = jnp.where(pick, mid, lo)
        hi = jnp.where(pick, hi, mid)
        return lo, hi

    lo, hi = jax.lax.fori_loop(0, _BITS, body, (lo, hi))
    return lo


def _attn_body(rand_ref, noise_ref, perturb_ref, out_ref):
    v = rand_ref[...]
    m = (v * _SCALE).astype(jnp.int32)
    thr = _row_threshold(m, K)
    sel = m >= thr
    blend = noise_ref[...] + perturb_ref[...]
    out_ref[...] = jnp.where(sel, blend, 1.0)


def _kqv_body(rand_ref, nk_ref, nq_ref, nv_ref, pk_ref, pq_ref, pv_ref,
              ok_ref, oq_ref, ov_ref):
    v = rand_ref[...]
    m = (v * _SCALE).astype(jnp.int32)
    thr = _row_threshold(m, K)
    sel = m >= thr
    ok_ref[...] = jnp.where(sel[:, 0:TOTAL], nk_ref[...] + pk_ref[...], 1.0)
    oq_ref[...] = jnp.where(sel[:, TOTAL:2 * TOTAL], nq_ref[...] + pq_ref[...], 1.0)
    ov_ref[...] = jnp.where(sel[:, 2 * TOTAL:3 * TOTAL], nv_ref[...] + pv_ref[...], 1.0)


def kernel(rand_attn, noise_attn, rand_kqv, noise_k, noise_q, noise_v,
           perturb_attn, perturb_k, perturb_q, perturb_v):
    R = 256
    grid = (BSZ // R,)

    row_spec = pl.BlockSpec((R, TOTAL), lambda i: (i, 0))
    kqv_spec = pl.BlockSpec((R, 3 * TOTAL), lambda i: (i, 0))
    p_spec = pl.BlockSpec((1, TOTAL), lambda i: (0, 0))

    attn_mask = pl.pallas_call(
        _attn_body,
        grid=grid,
        in_specs=[row_spec, row_spec, p_spec],
        out_specs=row_spec,
        out_shape=jax.ShapeDtypeStruct((BSZ, TOTAL), jnp.float32),
    )(rand_attn, noise_attn, perturb_attn.reshape(1, TOTAL))

    k_mask, q_mask, v_mask = pl.pallas_call(
        _kqv_body,
        grid=grid,
        in_specs=[kqv_spec, row_spec, row_spec, row_spec, p_spec, p_spec, p_spec],
        out_specs=[row_spec, row_spec, row_spec],
        out_shape=[jax.ShapeDtypeStruct((BSZ, TOTAL), jnp.float32)] * 3,
    )(rand_kqv, noise_k, noise_q, noise_v,
      perturb_k.reshape(1, TOTAL), perturb_q.reshape(1, TOTAL),
      perturb_v.reshape(1, TOTAL))

    shape = (BSZ, N_LAYERS, N_HEADS)
    return (attn_mask.reshape(shape), k_mask.reshape(shape),
            q_mask.reshape(shape), v_mask.reshape(shape))
